# Initial kernel scaffold; baseline (speedup 1.0000x reference)
import jax
import jax.numpy as jnp
from jax.experimental import pallas as pl


def kernel(p3, p4, p5, gt_boxes, gt_labels):
    raise NotImplementedError("write your pallas kernel here")


# speedup vs baseline: 5.1419x; 5.1419x over previous
import math

import jax
import jax.numpy as jnp
import numpy as np
from jax.experimental import pallas as pl

_NC = 80
_IMG = 640.0
_LAMBDA_BOX = 5.0
_LAMBDA_OBJ = 1.0
_LAMBDA_CLS = 0.5
_ASSIGN_CLS_W = 0.5
_CENTER_RADIUS = 2.0
_TOPK = 20
_CLS_SMOOTH = 0.05
_AREA_MIN = 4.0 / 1.25
_AREA_MAX = 256.0 * 1.25
_SIZE_PRIOR_W = 0.2
_AR_PRIOR_W = 0.1
_IOU_COST_W = 3.0
_CENTER_COST_W = 0.5
_EPS = 1e-7
_N = 8400
_NP = 8448
_M = 40


def _atan_pos(x):
    inv = x > 1.0
    z = jnp.where(inv, 1.0 / x, x)
    red = z > 0.4142135623730951
    w = jnp.where(red, (z - 1.0) / (z + 1.0), z)
    w2 = w * w
    p = w * (1.0 + w2 * (-1.0 / 3.0 + w2 * (1.0 / 5.0 + w2 * (
        -1.0 / 7.0 + w2 * (1.0 / 9.0 + w2 * (-1.0 / 11.0))))))
    p = jnp.where(red, 0.7853981633974483 + p, p)
    return jnp.where(inv, 1.5707963267948966 - p, p)


def _softplus(x):
    return jnp.maximum(x, 0.0) + jnp.log1p(jnp.exp(-jnp.abs(x)))


def _loss_body(anc_ref, boxraw_ref, obj_ref, cls_ref, gtf_ref, out_ref):
    b = pl.program_id(0)

    agx = anc_ref[0:1, :]
    agy = anc_ref[1:2, :]
    s = anc_ref[2:3, :]
    apx = anc_ref[3:4, :]
    apy = anc_ref[4:5, :]
    ls = anc_ref[5:6, :]

    braw = boxraw_ref[0]
    sx = jax.nn.sigmoid(braw[0:1, :])
    sy = jax.nn.sigmoid(braw[1:2, :])
    pcx = (sx * 2.0 - 0.5 + agx) * s
    pcy = (sy * 2.0 - 0.5 + agy) * s
    pw = _softplus(braw[2:3, :]) * s
    ph = _softplus(braw[3:4, :]) * s
    px1 = pcx - 0.5 * pw
    py1 = pcy - 0.5 * ph
    px2 = pcx + 0.5 * pw
    py2 = pcy + 0.5 * ph

    cls = cls_ref[0]
    cmax = jnp.max(cls, axis=0, keepdims=True)
    sse = jnp.sum(jnp.exp(cls - cmax), axis=0, keepdims=True)
    logz = jnp.log(sse) + cmax
    logp_rowsum = jnp.sum(cls, axis=0, keepdims=True) - float(_NC) * logz

    gtf = gtf_ref[0]
    gx1 = gtf[:, 0:1]
    gy1 = gtf[:, 1:2]
    gx2 = gtf[:, 2:3]
    gy2 = gtf[:, 3:4]
    labf = gtf[:, 4:5]
    gcx = (gx1 + gx2) * 0.5
    gcy = (gy1 + gy2) * 0.5
    gw = jnp.maximum(gx2 - gx1, _EPS)
    gh = jnp.maximum(gy2 - gy1, _EPS)

    cls_iota = jax.lax.broadcasted_iota(
        jnp.int32, (_M, _NC), 1).astype(jnp.float32)
    onehot = (cls_iota == labf).astype(jnp.float32)
    selcls = jnp.dot(onehot, cls, preferred_element_type=jnp.float32)
    cls_cost = logz - selcls

    ix1 = jnp.maximum(px1, gx1)
    iy1 = jnp.maximum(py1, gy1)
    ix2 = jnp.minimum(px2, gx2)
    iy2 = jnp.minimum(py2, gy2)
    inter = jnp.maximum(ix2 - ix1, 0.0) * jnp.maximum(iy2 - iy1, 0.0)
    pa = jnp.maximum(px2 - px1, 0.0) * jnp.maximum(py2 - py1, 0.0)
    ga = jnp.maximum(gx2 - gx1, 0.0) * jnp.maximum(gy2 - gy1, 0.0)
    iou = inter / (pa + ga - inter + _EPS)

    in_box = (apx > gx1) & (apx < gx2) & (apy > gy1) & (apy < gy2)
    r = _CENTER_RADIUS * s
    in_ctr = (jnp.abs(apx - gcx) < r) & (jnp.abs(apy - gcy) < r)
    cand = in_box & in_ctr

    dx = apx - gcx
    dy = apy - gcy
    cdist = jnp.sqrt(dx * dx + dy * dy + _EPS) / s

    lga = jnp.log(gw * gh)
    size_prior = _SIZE_PRIOR_W * (
        jax.nn.relu(lga - 2.0 * ls - math.log(_AREA_MAX))
        + jax.nn.relu(math.log(_AREA_MIN) - lga + 2.0 * ls))
    ar_prior = _AR_PRIOR_W * jnp.abs(jnp.log(gw / gh))

    cost = (_IOU_COST_W * (1.0 - iou) + _ASSIGN_CLS_W * cls_cost
            + _CENTER_COST_W * cdist + size_prior + ar_prior)
    cost = jnp.where(cand, cost, cost + 1e5)

    lane_iota = jax.lax.broadcasted_iota(
        jnp.int32, (_M, _NP), 1).astype(jnp.float32)
    costw = cost
    selw = jnp.zeros((_M, _NP), dtype=jnp.float32)
    for _ in range(_TOPK):
        mv = jnp.min(costw, axis=1, keepdims=True)
        midx = jnp.min(jnp.where(costw == mv, lane_iota, 1e9),
                       axis=1, keepdims=True)
        hit = (lane_iota == midx)
        selw = selw + hit.astype(jnp.float32)
        costw = jnp.where(hit, 1e30, costw)

    wf = selw * (cost < 1e4).astype(jnp.float32)
    npos = jnp.maximum(jnp.sum(wf, axis=(0, 1), keepdims=True), 1.0)

    pwc = jnp.maximum(px2 - px1, _EPS)
    phc = jnp.maximum(py2 - py1, _EPS)
    inter2 = (jnp.maximum(jnp.minimum(px2, gx2) - jnp.maximum(px1, gx1), 0.0)
              * jnp.maximum(jnp.minimum(py2, gy2) - jnp.maximum(py1, gy1), 0.0))
    union2 = pwc * phc + gw * gh - inter2 + _EPS
    iou2 = inter2 / union2
    ppx = (px1 + px2) * 0.5
    ppy = (py1 + py2) * 0.5
    cd = (ppx - gcx) ** 2 + (ppy - gcy) ** 2
    cw = jnp.maximum(px2, gx2) - jnp.minimum(px1, gx1)
    chh = jnp.maximum(py2, gy2) - jnp.minimum(py1, gy1)
    c2 = cw * cw + chh * chh + _EPS
    v = (4.0 / math.pi ** 2) * (_atan_pos(gw / gh) - _atan_pos(pwc / phc)) ** 2
    alpha = v / (v - iou2 + 1.0 + _EPS)
    ciou = iou2 - cd / c2 - alpha * v
    loss_box = jnp.sum(wf * (1.0 - ciou), axis=(0, 1), keepdims=True) / npos

    t_obj = jnp.max(wf, axis=0, keepdims=True)
    x = obj_ref[0]
    bce = jnp.maximum(x, 0.0) - x * t_obj + jnp.log1p(jnp.exp(-jnp.abs(x)))
    loss_obj = jnp.sum(bce, axis=(0, 1), keepdims=True) / float(_N)

    ce = ((1.0 - _CLS_SMOOTH) * cls_cost
          - (_CLS_SMOOTH / float(_NC)) * logp_rowsum)
    loss_cls = jnp.sum(wf * ce, axis=(0, 1), keepdims=True) / npos

    contrib = (_LAMBDA_BOX * loss_box + _LAMBDA_OBJ * loss_obj
               + _LAMBDA_CLS * loss_cls) / 8.0

    @pl.when(b == 0)
    def _():
        out_ref[:, :] = jnp.zeros((1, 1), jnp.float32)

    out_ref[:, :] += contrib


def _make_anchor_const():
    al = []
    sl = []
    for (h, w) in ((80, 80), (40, 40), (20, 20)):
        stride = 640.0 / max(h, w)
        sy, sx = np.meshgrid(np.arange(h), np.arange(w), indexing='ij')
        al.append(np.stack([sx, sy], axis=-1).astype(np.float32).reshape(-1, 2))
        sl.append(np.full((h * w,), stride, dtype=np.float32))
    anchors = np.concatenate(al, axis=0)
    strides = np.concatenate(sl, axis=0)
    anc = np.zeros((8, _NP), dtype=np.float32)
    anc[0, :_N] = anchors[:, 0]
    anc[1, :_N] = anchors[:, 1]
    anc[2, :_N] = strides
    anc[2, _N:] = 1.0
    anc[3, :_N] = (anchors[:, 0] + 0.5) * strides
    anc[3, _N:] = -1e6
    anc[4, :_N] = (anchors[:, 1] + 0.5) * strides
    anc[4, _N:] = -1e6
    anc[5, :_N] = np.log(strides)
    return jnp.asarray(anc)


_ANC = _make_anchor_const()


def kernel(p3, p4, p5, gt_boxes, gt_labels):
    B = p3.shape[0]
    pf = jnp.concatenate(
        [p.reshape(B, -1, 5 + _NC) for p in (p3, p4, p5)], axis=1)

    pad = _NP - _N
    boxraw = jnp.transpose(pf[..., 0:4], (0, 2, 1))
    boxraw = jnp.pad(boxraw, ((0, 0), (0, 0), (0, pad)))
    obj = jnp.transpose(pf[..., 4:5], (0, 2, 1))
    obj = jnp.pad(obj, ((0, 0), (0, 0), (0, pad)), constant_values=-100.0)
    cls = jnp.transpose(pf[..., 5:], (0, 2, 1))
    cls = jnp.pad(cls, ((0, 0), (0, 0), (0, pad)))

    gb = gt_boxes * _IMG
    gx1 = gb[..., 0] - 0.5 * gb[..., 2]
    gy1 = gb[..., 1] - 0.5 * gb[..., 3]
    gx2 = gb[..., 0] + 0.5 * gb[..., 2]
    gy2 = gb[..., 1] + 0.5 * gb[..., 3]
    labf = gt_labels.astype(jnp.float32)
    zero = jnp.zeros_like(labf)
    gtf = jnp.stack([gx1, gy1, gx2, gy2, labf, zero, zero, zero], axis=-1)

    out = pl.pallas_call(
        _loss_body,
        grid=(B,),
        in_specs=[
            pl.BlockSpec((8, _NP), lambda b: (0, 0)),
            pl.BlockSpec((1, 4, _NP), lambda b: (b, 0, 0)),
            pl.BlockSpec((1, 1, _NP), lambda b: (b, 0, 0)),
            pl.BlockSpec((1, _NC, _NP), lambda b: (b, 0, 0)),
            pl.BlockSpec((1, _M, 8), lambda b: (b, 0, 0)),
        ],
        out_specs=pl.BlockSpec((1, 1), lambda b: (0, 0)),
        out_shape=jax.ShapeDtypeStruct((1, 1), jnp.float32),
    )(_ANC, boxraw, obj, cls, gtf)
    return out[0, 0]


# speedup vs baseline: 6.6309x; 1.2896x over previous
import math

import jax
import jax.numpy as jnp
import numpy as np
from jax.experimental import pallas as pl

_NC = 80
_IMG = 640.0
_LAMBDA_BOX = 5.0
_LAMBDA_OBJ = 1.0
_LAMBDA_CLS = 0.5
_ASSIGN_CLS_W = 0.5
_CENTER_RADIUS = 2.0
_TOPK = 20
_CLS_SMOOTH = 0.05
_AREA_MIN = 4.0 / 1.25
_AREA_MAX = 256.0 * 1.25
_SIZE_PRIOR_W = 0.2
_AR_PRIOR_W = 0.1
_IOU_COST_W = 3.0
_CENTER_COST_W = 0.5
_EPS = 1e-7
_N = 8400
_NP = 8448
_M = 40


def _atan_pos(x):
    inv = x > 1.0
    z = jnp.where(inv, 1.0 / x, x)
    red = z > 0.4142135623730951
    w = jnp.where(red, (z - 1.0) / (z + 1.0), z)
    w2 = w * w
    p = w * (1.0 + w2 * (-1.0 / 3.0 + w2 * (1.0 / 5.0 + w2 * (
        -1.0 / 7.0 + w2 * (1.0 / 9.0 + w2 * (-1.0 / 11.0))))))
    p = jnp.where(red, 0.7853981633974483 + p, p)
    return jnp.where(inv, 1.5707963267948966 - p, p)


def _softplus(x):
    return jnp.maximum(x, 0.0) + jnp.log1p(jnp.exp(-jnp.abs(x)))


def _loss_body(anc_ref, boxraw_ref, obj_ref, cls_ref, gtf_ref, out_ref):
    b = pl.program_id(0)

    agx = anc_ref[0:1, :]
    agy = anc_ref[1:2, :]
    s = anc_ref[2:3, :]
    apx = anc_ref[3:4, :]
    apy = anc_ref[4:5, :]
    ls = anc_ref[5:6, :]

    braw = boxraw_ref[0]
    sx = jax.nn.sigmoid(braw[0:1, :])
    sy = jax.nn.sigmoid(braw[1:2, :])
    pcx = (sx * 2.0 - 0.5 + agx) * s
    pcy = (sy * 2.0 - 0.5 + agy) * s
    pw = _softplus(braw[2:3, :]) * s
    ph = _softplus(braw[3:4, :]) * s
    px1 = pcx - 0.5 * pw
    py1 = pcy - 0.5 * ph
    px2 = pcx + 0.5 * pw
    py2 = pcy + 0.5 * ph

    cls = cls_ref[0]
    cmax = jnp.max(cls, axis=0, keepdims=True)
    sse = jnp.sum(jnp.exp(cls - cmax), axis=0, keepdims=True)
    logz = jnp.log(sse) + cmax
    logp_rowsum = jnp.sum(cls, axis=0, keepdims=True) - float(_NC) * logz

    gtf = gtf_ref[0]
    gx1 = gtf[:, 0:1]
    gy1 = gtf[:, 1:2]
    gx2 = gtf[:, 2:3]
    gy2 = gtf[:, 3:4]
    labf = gtf[:, 4:5]
    gcx = (gx1 + gx2) * 0.5
    gcy = (gy1 + gy2) * 0.5
    gw = jnp.maximum(gx2 - gx1, _EPS)
    gh = jnp.maximum(gy2 - gy1, _EPS)

    cls_iota = jax.lax.broadcasted_iota(
        jnp.int32, (_M, _NC), 1).astype(jnp.float32)
    onehot = (cls_iota == labf).astype(jnp.float32)
    selcls = jnp.dot(onehot, cls, preferred_element_type=jnp.float32)
    cls_cost = logz - selcls

    ix1 = jnp.maximum(px1, gx1)
    iy1 = jnp.maximum(py1, gy1)
    ix2 = jnp.minimum(px2, gx2)
    iy2 = jnp.minimum(py2, gy2)
    inter = jnp.maximum(ix2 - ix1, 0.0) * jnp.maximum(iy2 - iy1, 0.0)
    pa = jnp.maximum(px2 - px1, 0.0) * jnp.maximum(py2 - py1, 0.0)
    ga = jnp.maximum(gx2 - gx1, 0.0) * jnp.maximum(gy2 - gy1, 0.0)
    iou = inter / (pa + ga - inter + _EPS)

    in_box = (apx > gx1) & (apx < gx2) & (apy > gy1) & (apy < gy2)
    r = _CENTER_RADIUS * s
    in_ctr = (jnp.abs(apx - gcx) < r) & (jnp.abs(apy - gcy) < r)
    cand = in_box & in_ctr

    dx = apx - gcx
    dy = apy - gcy
    inv_s = anc_ref[6:7, :]
    cdist = jnp.sqrt(dx * dx + dy * dy + _EPS) / s

    lga = jnp.log(gw * gh)
    size_prior = _SIZE_PRIOR_W * (
        jax.nn.relu(lga - 2.0 * ls - math.log(_AREA_MAX))
        + jax.nn.relu(math.log(_AREA_MIN) - lga + 2.0 * ls))
    ar_prior = _AR_PRIOR_W * jnp.abs(jnp.log(gw / gh))

    cost = (_IOU_COST_W * (1.0 - iou) + _ASSIGN_CLS_W * cls_cost
            + _CENTER_COST_W * cdist + size_prior + ar_prior)

    costw = jnp.where(cand, cost, 3e30)
    for _ in range(_TOPK):
        mv = jnp.min(costw, axis=1, keepdims=True)
        costw = jnp.where(costw == mv, 1e30, costw)

    wf = ((costw > 1e29) & (cost < 1e4) & cand).astype(jnp.float32)
    npos = jnp.maximum(jnp.sum(wf, axis=(0, 1), keepdims=True), 1.0)

    pwc = jnp.maximum(px2 - px1, _EPS)
    phc = jnp.maximum(py2 - py1, _EPS)
    inter2 = (jnp.maximum(jnp.minimum(px2, gx2) - jnp.maximum(px1, gx1), 0.0)
              * jnp.maximum(jnp.minimum(py2, gy2) - jnp.maximum(py1, gy1), 0.0))
    union2 = pwc * phc + gw * gh - inter2 + _EPS
    iou2 = inter2 / union2
    ppx = (px1 + px2) * 0.5
    ppy = (py1 + py2) * 0.5
    cd = (ppx - gcx) ** 2 + (ppy - gcy) ** 2
    cw = jnp.maximum(px2, gx2) - jnp.minimum(px1, gx1)
    chh = jnp.maximum(py2, gy2) - jnp.minimum(py1, gy1)
    c2 = cw * cw + chh * chh + _EPS
    v = (4.0 / math.pi ** 2) * (_atan_pos(gw / gh) - _atan_pos(pwc / phc)) ** 2
    alpha = v / (v - iou2 + 1.0 + _EPS)
    ciou = iou2 - cd / c2 - alpha * v
    loss_box = jnp.sum(wf * (1.0 - ciou), axis=(0, 1), keepdims=True) / npos

    t_obj = jnp.max(wf, axis=0, keepdims=True)
    x = obj_ref[0]
    bce = jnp.maximum(x, 0.0) - x * t_obj + jnp.log1p(jnp.exp(-jnp.abs(x)))
    loss_obj = jnp.sum(bce, axis=(0, 1), keepdims=True) / float(_N)

    ce = ((1.0 - _CLS_SMOOTH) * cls_cost
          - (_CLS_SMOOTH / float(_NC)) * logp_rowsum)
    loss_cls = jnp.sum(wf * ce, axis=(0, 1), keepdims=True) / npos

    contrib = (_LAMBDA_BOX * loss_box + _LAMBDA_OBJ * loss_obj
               + _LAMBDA_CLS * loss_cls) / 8.0

    @pl.when(b == 0)
    def _():
        out_ref[:, :] = jnp.zeros((1, 1), jnp.float32)

    out_ref[:, :] += contrib


def _make_anchor_const():
    al = []
    sl = []
    for (h, w) in ((80, 80), (40, 40), (20, 20)):
        stride = 640.0 / max(h, w)
        sy, sx = np.meshgrid(np.arange(h), np.arange(w), indexing='ij')
        al.append(np.stack([sx, sy], axis=-1).astype(np.float32).reshape(-1, 2))
        sl.append(np.full((h * w,), stride, dtype=np.float32))
    anchors = np.concatenate(al, axis=0)
    strides = np.concatenate(sl, axis=0)
    anc = np.zeros((8, _NP), dtype=np.float32)
    anc[0, :_N] = anchors[:, 0]
    anc[1, :_N] = anchors[:, 1]
    anc[2, :_N] = strides
    anc[2, _N:] = 1.0
    anc[3, :_N] = (anchors[:, 0] + 0.5) * strides
    anc[3, _N:] = -1e6
    anc[4, :_N] = (anchors[:, 1] + 0.5) * strides
    anc[4, _N:] = -1e6
    anc[5, :_N] = np.log(strides)
    anc[6, :_N] = 1.0 / strides
    anc[6, _N:] = 1.0
    return anc


_ANC_NP = _make_anchor_const()


def kernel(p3, p4, p5, gt_boxes, gt_labels):
    B = p3.shape[0]
    pf = jnp.concatenate(
        [p.reshape(B, -1, 5 + _NC) for p in (p3, p4, p5)], axis=1)

    pad = _NP - _N
    boxraw = jnp.transpose(pf[..., 0:4], (0, 2, 1))
    boxraw = jnp.pad(boxraw, ((0, 0), (0, 0), (0, pad)))
    obj = jnp.transpose(pf[..., 4:5], (0, 2, 1))
    obj = jnp.pad(obj, ((0, 0), (0, 0), (0, pad)), constant_values=-100.0)
    cls = jnp.transpose(pf[..., 5:], (0, 2, 1))
    cls = jnp.pad(cls, ((0, 0), (0, 0), (0, pad)))

    gb = gt_boxes * _IMG
    gx1 = gb[..., 0] - 0.5 * gb[..., 2]
    gy1 = gb[..., 1] - 0.5 * gb[..., 3]
    gx2 = gb[..., 0] + 0.5 * gb[..., 2]
    gy2 = gb[..., 1] + 0.5 * gb[..., 3]
    labf = gt_labels.astype(jnp.float32)
    zero = jnp.zeros_like(labf)
    gtf = jnp.stack([gx1, gy1, gx2, gy2, labf, zero, zero, zero], axis=-1)

    out = pl.pallas_call(
        _loss_body,
        grid=(B,),
        in_specs=[
            pl.BlockSpec((8, _NP), lambda b: (0, 0)),
            pl.BlockSpec((1, 4, _NP), lambda b: (b, 0, 0)),
            pl.BlockSpec((1, 1, _NP), lambda b: (b, 0, 0)),
            pl.BlockSpec((1, _NC, _NP), lambda b: (b, 0, 0)),
            pl.BlockSpec((1, _M, 8), lambda b: (b, 0, 0)),
        ],
        out_specs=pl.BlockSpec((1, 1), lambda b: (0, 0)),
        out_shape=jax.ShapeDtypeStruct((1, 1), jnp.float32),
    )(jnp.asarray(_ANC_NP), boxraw, obj, cls, gtf)
    return out[0, 0]


# speedup vs baseline: 11.1651x; 1.6838x over previous
import math

import jax
import jax.numpy as jnp
import numpy as np
from jax.experimental import pallas as pl

_NC = 80
_IMG = 640.0
_LAMBDA_BOX = 5.0
_LAMBDA_OBJ = 1.0
_LAMBDA_CLS = 0.5
_ASSIGN_CLS_W = 0.5
_CENTER_RADIUS = 2.0
_TOPK = 20
_CLS_SMOOTH = 0.05
_AREA_MIN = 4.0 / 1.25
_AREA_MAX = 256.0 * 1.25
_SIZE_PRIOR_W = 0.2
_AR_PRIOR_W = 0.1
_IOU_COST_W = 3.0
_CENTER_COST_W = 0.5
_EPS = 1e-7
_N = 8400
_M = 40


def _atan_pos(x):
    inv = x > 1.0
    z = jnp.where(inv, 1.0 / x, x)
    red = z > 0.4142135623730951
    w = jnp.where(red, (z - 1.0) / (z + 1.0), z)
    w2 = w * w
    p = w * (1.0 + w2 * (-1.0 / 3.0 + w2 * (1.0 / 5.0 + w2 * (
        -1.0 / 7.0 + w2 * (1.0 / 9.0 + w2 * (-1.0 / 11.0))))))
    p = jnp.where(red, 0.7853981633974483 + p, p)
    return jnp.where(inv, 1.5707963267948966 - p, p)


def _softplus(x):
    return jnp.maximum(x, 0.0) + jnp.log1p(jnp.exp(-jnp.abs(x)))


def _loss_body(anc_ref, p3_ref, p4_ref, p5_ref, gtf_ref, out_ref):
    b = pl.program_id(0)

    agx = anc_ref[0:1, :]
    agy = anc_ref[1:2, :]
    s = anc_ref[2:3, :]
    apx = anc_ref[3:4, :]
    apy = anc_ref[4:5, :]
    ls = anc_ref[5:6, :]
    inv_s = anc_ref[6:7, :]

    nat = jnp.concatenate([p3_ref[0], p4_ref[0], p5_ref[0]], axis=0)
    pft = jnp.transpose(nat, (1, 0))

    braw = pft[0:4, :]
    sx = jax.nn.sigmoid(braw[0:1, :])
    sy = jax.nn.sigmoid(braw[1:2, :])
    pcx = (sx * 2.0 - 0.5 + agx) * s
    pcy = (sy * 2.0 - 0.5 + agy) * s
    pw = _softplus(braw[2:3, :]) * s
    ph = _softplus(braw[3:4, :]) * s
    px1 = pcx - 0.5 * pw
    py1 = pcy - 0.5 * ph
    px2 = pcx + 0.5 * pw
    py2 = pcy + 0.5 * ph

    cls = pft[5:85, :]
    cmax = jnp.max(cls, axis=0, keepdims=True)
    sse = jnp.sum(jnp.exp(cls - cmax), axis=0, keepdims=True)
    logz = jnp.log(sse) + cmax
    logp_rowsum = jnp.sum(cls, axis=0, keepdims=True) - float(_NC) * logz

    gtf = gtf_ref[0]
    gx1 = gtf[:, 0:1]
    gy1 = gtf[:, 1:2]
    gx2 = gtf[:, 2:3]
    gy2 = gtf[:, 3:4]
    labf = gtf[:, 4:5]
    gcx = (gx1 + gx2) * 0.5
    gcy = (gy1 + gy2) * 0.5
    gw = jnp.maximum(gx2 - gx1, _EPS)
    gh = jnp.maximum(gy2 - gy1, _EPS)

    cls_iota = jax.lax.broadcasted_iota(
        jnp.int32, (_M, _NC), 1).astype(jnp.float32)
    onehot = (cls_iota == labf).astype(jnp.float32)
    selcls = jnp.dot(onehot, cls, preferred_element_type=jnp.float32)
    cls_cost = logz - selcls

    ix1 = jnp.maximum(px1, gx1)
    iy1 = jnp.maximum(py1, gy1)
    ix2 = jnp.minimum(px2, gx2)
    iy2 = jnp.minimum(py2, gy2)
    inter = jnp.maximum(ix2 - ix1, 0.0) * jnp.maximum(iy2 - iy1, 0.0)
    pa = jnp.maximum(px2 - px1, 0.0) * jnp.maximum(py2 - py1, 0.0)
    ga = jnp.maximum(gx2 - gx1, 0.0) * jnp.maximum(gy2 - gy1, 0.0)
    iou = inter / (pa + ga - inter + _EPS)

    in_box = (apx > gx1) & (apx < gx2) & (apy > gy1) & (apy < gy2)
    r = _CENTER_RADIUS * s
    in_ctr = (jnp.abs(apx - gcx) < r) & (jnp.abs(apy - gcy) < r)
    cand = in_box & in_ctr

    dx = apx - gcx
    dy = apy - gcy
    cdist = jnp.sqrt(dx * dx + dy * dy + _EPS) * inv_s

    lga = jnp.log(gw * gh)
    size_prior = _SIZE_PRIOR_W * (
        jax.nn.relu(lga - 2.0 * ls - math.log(_AREA_MAX))
        + jax.nn.relu(math.log(_AREA_MIN) - lga + 2.0 * ls))
    ar_prior = _AR_PRIOR_W * jnp.abs(jnp.log(gw / gh))

    cost = (_IOU_COST_W * (1.0 - iou) + _ASSIGN_CLS_W * cls_cost
            + _CENTER_COST_W * cdist + size_prior + ar_prior)

    costw = jnp.where(cand, cost, 3e30)
    for _ in range(_TOPK):
        mv = jnp.min(costw, axis=1, keepdims=True)
        costw = jnp.where(costw == mv, 1e30, costw)

    wf = ((costw > 1e29) & (cost < 1e4) & cand).astype(jnp.float32)
    npos = jnp.maximum(jnp.sum(wf, axis=(0, 1), keepdims=True), 1.0)

    pwc = jnp.maximum(px2 - px1, _EPS)
    phc = jnp.maximum(py2 - py1, _EPS)
    inter2 = (jnp.maximum(jnp.minimum(px2, gx2) - jnp.maximum(px1, gx1), 0.0)
              * jnp.maximum(jnp.minimum(py2, gy2) - jnp.maximum(py1, gy1), 0.0))
    union2 = pwc * phc + gw * gh - inter2 + _EPS
    iou2 = inter2 / union2
    ppx = (px1 + px2) * 0.5
    ppy = (py1 + py2) * 0.5
    cd = (ppx - gcx) ** 2 + (ppy - gcy) ** 2
    cw = jnp.maximum(px2, gx2) - jnp.minimum(px1, gx1)
    chh = jnp.maximum(py2, gy2) - jnp.minimum(py1, gy1)
    c2 = cw * cw + chh * chh + _EPS
    v = (4.0 / math.pi ** 2) * (_atan_pos(gw / gh) - _atan_pos(pwc / phc)) ** 2
    alpha = v / (v - iou2 + 1.0 + _EPS)
    ciou = iou2 - cd / c2 - alpha * v
    loss_box = jnp.sum(wf * (1.0 - ciou), axis=(0, 1), keepdims=True) / npos

    t_obj = jnp.max(wf, axis=0, keepdims=True)
    x = pft[4:5, :]
    bce = jnp.maximum(x, 0.0) - x * t_obj + jnp.log1p(jnp.exp(-jnp.abs(x)))
    loss_obj = jnp.sum(bce, axis=(0, 1), keepdims=True) / float(_N)

    ce = ((1.0 - _CLS_SMOOTH) * cls_cost
          - (_CLS_SMOOTH / float(_NC)) * logp_rowsum)
    loss_cls = jnp.sum(wf * ce, axis=(0, 1), keepdims=True) / npos

    contrib = (_LAMBDA_BOX * loss_box + _LAMBDA_OBJ * loss_obj
               + _LAMBDA_CLS * loss_cls) / 8.0

    @pl.when(b == 0)
    def _():
        out_ref[:, :] = jnp.zeros((1, 1), jnp.float32)

    out_ref[:, :] += contrib


def _make_anchor_const():
    al = []
    sl = []
    for (h, w) in ((80, 80), (40, 40), (20, 20)):
        stride = 640.0 / max(h, w)
        sy, sx = np.meshgrid(np.arange(h), np.arange(w), indexing='ij')
        al.append(np.stack([sx, sy], axis=-1).astype(np.float32).reshape(-1, 2))
        sl.append(np.full((h * w,), stride, dtype=np.float32))
    anchors = np.concatenate(al, axis=0)
    strides = np.concatenate(sl, axis=0)
    anc = np.zeros((8, _N), dtype=np.float32)
    anc[0] = anchors[:, 0]
    anc[1] = anchors[:, 1]
    anc[2] = strides
    anc[3] = (anchors[:, 0] + 0.5) * strides
    anc[4] = (anchors[:, 1] + 0.5) * strides
    anc[5] = np.log(strides)
    anc[6] = 1.0 / strides
    return anc


_ANC_NP = _make_anchor_const()


def kernel(p3, p4, p5, gt_boxes, gt_labels):
    B = p3.shape[0]
    f3 = p3.reshape(B, -1, 5 + _NC)
    f4 = p4.reshape(B, -1, 5 + _NC)
    f5 = p5.reshape(B, -1, 5 + _NC)

    gb = gt_boxes * _IMG
    gx1 = gb[..., 0] - 0.5 * gb[..., 2]
    gy1 = gb[..., 1] - 0.5 * gb[..., 3]
    gx2 = gb[..., 0] + 0.5 * gb[..., 2]
    gy2 = gb[..., 1] + 0.5 * gb[..., 3]
    labf = gt_labels.astype(jnp.float32)
    zero = jnp.zeros_like(labf)
    gtf = jnp.stack([gx1, gy1, gx2, gy2, labf, zero, zero, zero], axis=-1)

    out = pl.pallas_call(
        _loss_body,
        grid=(B,),
        in_specs=[
            pl.BlockSpec((8, _N), lambda b: (0, 0)),
            pl.BlockSpec((1, 6400, 85), lambda b: (b, 0, 0)),
            pl.BlockSpec((1, 1600, 85), lambda b: (b, 0, 0)),
            pl.BlockSpec((1, 400, 85), lambda b: (b, 0, 0)),
            pl.BlockSpec((1, _M, 8), lambda b: (b, 0, 0)),
        ],
        out_specs=pl.BlockSpec((1, 1), lambda b: (0, 0)),
        out_shape=jax.ShapeDtypeStruct((1, 1), jnp.float32),
    )(jnp.asarray(_ANC_NP), f3, f4, f5, gtf)
    return out[0, 0]


# speedup vs baseline: 11.2152x; 1.0045x over previous
import math

import jax
import jax.numpy as jnp
import numpy as np
from jax.experimental import pallas as pl

_NC = 80
_IMG = 640.0
_LAMBDA_BOX = 5.0
_LAMBDA_OBJ = 1.0
_LAMBDA_CLS = 0.5
_ASSIGN_CLS_W = 0.5
_CENTER_RADIUS = 2.0
_TOPK = 20
_CLS_SMOOTH = 0.05
_AREA_MIN = 4.0 / 1.25
_AREA_MAX = 256.0 * 1.25
_SIZE_PRIOR_W = 0.2
_AR_PRIOR_W = 0.1
_IOU_COST_W = 3.0
_CENTER_COST_W = 0.5
_EPS = 1e-7
_N = 8400
_M = 40


def _atan_pos(x):
    inv = x > 1.0
    z = jnp.where(inv, 1.0 / x, x)
    red = z > 0.4142135623730951
    w = jnp.where(red, (z - 1.0) / (z + 1.0), z)
    w2 = w * w
    p = w * (1.0 + w2 * (-1.0 / 3.0 + w2 * (1.0 / 5.0 + w2 * (
        -1.0 / 7.0 + w2 * (1.0 / 9.0 + w2 * (-1.0 / 11.0))))))
    p = jnp.where(red, 0.7853981633974483 + p, p)
    return jnp.where(inv, 1.5707963267948966 - p, p)


def _softplus(x):
    return jnp.maximum(x, 0.0) + jnp.log1p(jnp.exp(-jnp.abs(x)))


def _loss_body(anc_ref, p3_ref, p4_ref, p5_ref, gtf_ref, out_ref):
    b = pl.program_id(0)

    agx = anc_ref[0:1, :]
    agy = anc_ref[1:2, :]
    s = anc_ref[2:3, :]
    apx = anc_ref[3:4, :]
    apy = anc_ref[4:5, :]
    ls = anc_ref[5:6, :]
    inv_s = anc_ref[6:7, :]

    nat = jnp.concatenate([p3_ref[0], p4_ref[0], p5_ref[0]], axis=0)
    pft = jnp.transpose(nat, (1, 0))

    braw = pft[0:4, :]
    sx = jax.nn.sigmoid(braw[0:1, :])
    sy = jax.nn.sigmoid(braw[1:2, :])
    pcx = (sx * 2.0 - 0.5 + agx) * s
    pcy = (sy * 2.0 - 0.5 + agy) * s
    pw = _softplus(braw[2:3, :]) * s
    ph = _softplus(braw[3:4, :]) * s
    px1 = pcx - 0.5 * pw
    py1 = pcy - 0.5 * ph
    px2 = pcx + 0.5 * pw
    py2 = pcy + 0.5 * ph

    cls = pft[5:85, :]
    cmax = jnp.max(cls, axis=0, keepdims=True)
    sse = jnp.sum(jnp.exp(cls - cmax), axis=0, keepdims=True)
    logz = jnp.log(sse) + cmax
    logp_rowsum = jnp.sum(cls, axis=0, keepdims=True) - float(_NC) * logz

    gtf = gtf_ref[0]
    gx1 = gtf[:, 0:1]
    gy1 = gtf[:, 1:2]
    gx2 = gtf[:, 2:3]
    gy2 = gtf[:, 3:4]
    labf = gtf[:, 4:5]
    gcx = (gx1 + gx2) * 0.5
    gcy = (gy1 + gy2) * 0.5
    gw = jnp.maximum(gx2 - gx1, _EPS)
    gh = jnp.maximum(gy2 - gy1, _EPS)

    cls_iota = jax.lax.broadcasted_iota(
        jnp.int32, (_M, _NC), 1).astype(jnp.float32)
    onehot = (cls_iota == labf).astype(jnp.float32)
    selcls = jnp.dot(onehot, cls, preferred_element_type=jnp.float32)
    cls_cost = logz - selcls

    ix1 = jnp.maximum(px1, gx1)
    iy1 = jnp.maximum(py1, gy1)
    ix2 = jnp.minimum(px2, gx2)
    iy2 = jnp.minimum(py2, gy2)
    inter = jnp.maximum(ix2 - ix1, 0.0) * jnp.maximum(iy2 - iy1, 0.0)
    pa = jnp.maximum(px2 - px1, 0.0) * jnp.maximum(py2 - py1, 0.0)
    ga = jnp.maximum(gx2 - gx1, 0.0) * jnp.maximum(gy2 - gy1, 0.0)
    iou = inter / (pa + ga - inter + _EPS)

    in_box = (apx > gx1) & (apx < gx2) & (apy > gy1) & (apy < gy2)
    r = _CENTER_RADIUS * s
    in_ctr = (jnp.abs(apx - gcx) < r) & (jnp.abs(apy - gcy) < r)
    cand = in_box & in_ctr

    dx = apx - gcx
    dy = apy - gcy
    cdist = jnp.sqrt(dx * dx + dy * dy + _EPS) * inv_s

    lga = jnp.log(gw * gh)
    size_prior = _SIZE_PRIOR_W * (
        jax.nn.relu(lga - 2.0 * ls - math.log(_AREA_MAX))
        + jax.nn.relu(math.log(_AREA_MIN) - lga + 2.0 * ls))
    ar_prior = _AR_PRIOR_W * jnp.abs(jnp.log(gw / gh))

    cost = (_IOU_COST_W * (1.0 - iou) + _ASSIGN_CLS_W * cls_cost
            + _CENTER_COST_W * cdist + size_prior + ar_prior)

    costw = jnp.where(cand, cost, 3e30)
    mv = jnp.min(costw, axis=1, keepdims=True)
    for _ in range(_TOPK - 1):
        mv = jnp.min(jnp.where(costw > mv, costw, 3e30),
                     axis=1, keepdims=True)

    wf = ((cost <= mv) & (cost < 1e4) & cand).astype(jnp.float32)
    npos = jnp.maximum(jnp.sum(wf, axis=(0, 1), keepdims=True), 1.0)

    pwc = jnp.maximum(px2 - px1, _EPS)
    phc = jnp.maximum(py2 - py1, _EPS)
    union2 = pwc * phc + gw * gh - inter + _EPS
    iou2 = inter / union2
    ppx = (px1 + px2) * 0.5
    ppy = (py1 + py2) * 0.5
    cd = (ppx - gcx) ** 2 + (ppy - gcy) ** 2
    cw = jnp.maximum(px2, gx2) - jnp.minimum(px1, gx1)
    chh = jnp.maximum(py2, gy2) - jnp.minimum(py1, gy1)
    c2 = cw * cw + chh * chh + _EPS
    v = (4.0 / math.pi ** 2) * (_atan_pos(gw / gh) - _atan_pos(pwc / phc)) ** 2
    alpha = v / (v - iou2 + 1.0 + _EPS)
    ciou = iou2 - cd / c2 - alpha * v
    loss_box = jnp.sum(wf * (1.0 - ciou), axis=(0, 1), keepdims=True) / npos

    t_obj = jnp.max(wf, axis=0, keepdims=True)
    x = pft[4:5, :]
    bce = jnp.maximum(x, 0.0) - x * t_obj + jnp.log1p(jnp.exp(-jnp.abs(x)))
    loss_obj = jnp.sum(bce, axis=(0, 1), keepdims=True) / float(_N)

    ce = ((1.0 - _CLS_SMOOTH) * cls_cost
          - (_CLS_SMOOTH / float(_NC)) * logp_rowsum)
    loss_cls = jnp.sum(wf * ce, axis=(0, 1), keepdims=True) / npos

    contrib = (_LAMBDA_BOX * loss_box + _LAMBDA_OBJ * loss_obj
               + _LAMBDA_CLS * loss_cls) / 8.0

    @pl.when(b == 0)
    def _():
        out_ref[:, :] = jnp.zeros((1, 1), jnp.float32)

    out_ref[:, :] += contrib


def _make_anchor_const():
    al = []
    sl = []
    for (h, w) in ((80, 80), (40, 40), (20, 20)):
        stride = 640.0 / max(h, w)
        sy, sx = np.meshgrid(np.arange(h), np.arange(w), indexing='ij')
        al.append(np.stack([sx, sy], axis=-1).astype(np.float32).reshape(-1, 2))
        sl.append(np.full((h * w,), stride, dtype=np.float32))
    anchors = np.concatenate(al, axis=0)
    strides = np.concatenate(sl, axis=0)
    anc = np.zeros((8, _N), dtype=np.float32)
    anc[0] = anchors[:, 0]
    anc[1] = anchors[:, 1]
    anc[2] = strides
    anc[3] = (anchors[:, 0] + 0.5) * strides
    anc[4] = (anchors[:, 1] + 0.5) * strides
    anc[5] = np.log(strides)
    anc[6] = 1.0 / strides
    return anc


_ANC_NP = _make_anchor_const()


def kernel(p3, p4, p5, gt_boxes, gt_labels):
    B = p3.shape[0]
    f3 = p3.reshape(B, -1, 5 + _NC)
    f4 = p4.reshape(B, -1, 5 + _NC)
    f5 = p5.reshape(B, -1, 5 + _NC)

    gb = gt_boxes * _IMG
    gx1 = gb[..., 0] - 0.5 * gb[..., 2]
    gy1 = gb[..., 1] - 0.5 * gb[..., 3]
    gx2 = gb[..., 0] + 0.5 * gb[..., 2]
    gy2 = gb[..., 1] + 0.5 * gb[..., 3]
    labf = gt_labels.astype(jnp.float32)
    zero = jnp.zeros_like(labf)
    gtf = jnp.stack([gx1, gy1, gx2, gy2, labf, zero, zero, zero], axis=-1)

    out = pl.pallas_call(
        _loss_body,
        grid=(B,),
        in_specs=[
            pl.BlockSpec((8, _N), lambda b: (0, 0)),
            pl.BlockSpec((1, 6400, 85), lambda b: (b, 0, 0)),
            pl.BlockSpec((1, 1600, 85), lambda b: (b, 0, 0)),
            pl.BlockSpec((1, 400, 85), lambda b: (b, 0, 0)),
            pl.BlockSpec((1, _M, 8), lambda b: (b, 0, 0)),
        ],
        out_specs=pl.BlockSpec((1, 1), lambda b: (0, 0)),
        out_shape=jax.ShapeDtypeStruct((1, 1), jnp.float32),
    )(jnp.asarray(_ANC_NP), f3, f4, f5, gtf)
    return out[0, 0]


# speedup vs baseline: 11.4109x; 1.0174x over previous
import math

import jax
import jax.numpy as jnp
import numpy as np
from jax.experimental import pallas as pl

_NC = 80
_IMG = 640.0
_LAMBDA_BOX = 5.0
_LAMBDA_OBJ = 1.0
_LAMBDA_CLS = 0.5
_ASSIGN_CLS_W = 0.5
_CENTER_RADIUS = 2.0
_TOPK = 20
_CLS_SMOOTH = 0.05
_AREA_MIN = 4.0 / 1.25
_AREA_MAX = 256.0 * 1.25
_SIZE_PRIOR_W = 0.2
_AR_PRIOR_W = 0.1
_IOU_COST_W = 3.0
_CENTER_COST_W = 0.5
_EPS = 1e-7
_N = 8400
_M = 40


def _atan_pos(x):
    inv = x > 1.0
    z = jnp.where(inv, 1.0 / x, x)
    red = z > 0.4142135623730951
    w = jnp.where(red, (z - 1.0) / (z + 1.0), z)
    w2 = w * w
    p = w * (1.0 + w2 * (-1.0 / 3.0 + w2 * (1.0 / 5.0 + w2 * (
        -1.0 / 7.0 + w2 * (1.0 / 9.0 + w2 * (-1.0 / 11.0))))))
    p = jnp.where(red, 0.7853981633974483 + p, p)
    return jnp.where(inv, 1.5707963267948966 - p, p)


def _softplus(x):
    return jnp.maximum(x, 0.0) + jnp.log1p(jnp.exp(-jnp.abs(x)))


def _loss_body(anc_ref, p3_ref, p4_ref, p5_ref, gtf_ref, out_ref):
    b = pl.program_id(0)

    agx = anc_ref[0:1, :]
    agy = anc_ref[1:2, :]
    s = anc_ref[2:3, :]
    apx = anc_ref[3:4, :]
    apy = anc_ref[4:5, :]
    ls = anc_ref[5:6, :]
    inv_s = anc_ref[6:7, :]

    nat = jnp.concatenate([p3_ref[0], p4_ref[0], p5_ref[0]], axis=0)
    pft = jnp.transpose(nat, (1, 0))

    braw = pft[0:4, :]
    sx = jax.nn.sigmoid(braw[0:1, :])
    sy = jax.nn.sigmoid(braw[1:2, :])
    pcx = (sx * 2.0 - 0.5 + agx) * s
    pcy = (sy * 2.0 - 0.5 + agy) * s
    pw = _softplus(braw[2:3, :]) * s
    ph = _softplus(braw[3:4, :]) * s
    px1 = pcx - 0.5 * pw
    py1 = pcy - 0.5 * ph
    px2 = pcx + 0.5 * pw
    py2 = pcy + 0.5 * ph

    cls = pft[5:85, :]
    cmax = jnp.max(cls, axis=0, keepdims=True)
    sse = jnp.sum(jnp.exp(cls - cmax), axis=0, keepdims=True)
    logz = jnp.log(sse) + cmax
    logp_rowsum = jnp.sum(cls, axis=0, keepdims=True) - float(_NC) * logz

    gtf = gtf_ref[0]
    gx1 = gtf[:, 0:1]
    gy1 = gtf[:, 1:2]
    gx2 = gtf[:, 2:3]
    gy2 = gtf[:, 3:4]
    labf = gtf[:, 4:5]
    gcx = (gx1 + gx2) * 0.5
    gcy = (gy1 + gy2) * 0.5
    gw = jnp.maximum(gx2 - gx1, _EPS)
    gh = jnp.maximum(gy2 - gy1, _EPS)

    cls_iota = jax.lax.broadcasted_iota(
        jnp.int32, (_M, _NC), 1).astype(jnp.float32)
    onehot = jnp.where(cls_iota == labf, _ASSIGN_CLS_W, 0.0)
    selcls05 = jnp.dot(onehot, cls, preferred_element_type=jnp.float32)
    cc05 = _ASSIGN_CLS_W * logz - selcls05

    ix1 = jnp.maximum(px1, gx1)
    iy1 = jnp.maximum(py1, gy1)
    ix2 = jnp.minimum(px2, gx2)
    iy2 = jnp.minimum(py2, gy2)
    inter = jnp.maximum(ix2 - ix1, 0.0) * jnp.maximum(iy2 - iy1, 0.0)
    pa = jnp.maximum(px2 - px1, 0.0) * jnp.maximum(py2 - py1, 0.0)
    ga = jnp.maximum(gx2 - gx1, 0.0) * jnp.maximum(gy2 - gy1, 0.0)
    iou = inter / (pa + ga - inter + _EPS)

    in_box = (apx > gx1) & (apx < gx2) & (apy > gy1) & (apy < gy2)
    r = _CENTER_RADIUS * s
    in_ctr = (jnp.abs(apx - gcx) < r) & (jnp.abs(apy - gcy) < r)
    cand = in_box & in_ctr

    dx = apx - gcx
    dy = apy - gcy
    cdist05 = jnp.sqrt(dx * dx + dy * dy + _EPS) * inv_s

    lga = jnp.log(gw * gh)
    size_prior = _SIZE_PRIOR_W * (
        jax.nn.relu(lga - 2.0 * ls - math.log(_AREA_MAX))
        + jax.nn.relu(math.log(_AREA_MIN) - lga + 2.0 * ls))
    ar_prior = _AR_PRIOR_W * jnp.abs(jnp.log(gw / gh))

    cost = (_IOU_COST_W * (1.0 - iou) + cc05
            + cdist05 + size_prior + ar_prior)

    costw = jnp.where(cand, cost, 3e30)
    mv = jnp.min(costw, axis=1, keepdims=True)
    for _ in range(_TOPK - 1):
        mv = jnp.min(jnp.where(costw > mv, costw, 3e30),
                     axis=1, keepdims=True)

    wf = ((cost <= mv) & (cost < 1e4) & cand).astype(jnp.float32)
    colsum_wf = jnp.sum(wf, axis=0, keepdims=True)
    npos_raw = jnp.sum(colsum_wf, axis=(0, 1), keepdims=True)
    npos = jnp.maximum(npos_raw, 1.0)

    pwc = jnp.maximum(px2 - px1, _EPS)
    phc = jnp.maximum(py2 - py1, _EPS)
    union2 = pwc * phc + gw * gh - inter + _EPS
    iou2 = inter / union2
    ppx = (px1 + px2) * 0.5
    ppy = (py1 + py2) * 0.5
    cd = (ppx - gcx) ** 2 + (ppy - gcy) ** 2
    cw = jnp.maximum(px2, gx2) - jnp.minimum(px1, gx1)
    chh = jnp.maximum(py2, gy2) - jnp.minimum(py1, gy1)
    c2 = cw * cw + chh * chh + _EPS
    v = (4.0 / math.pi ** 2) * (_atan_pos(gw / gh) - _atan_pos(pwc / phc)) ** 2
    alpha = v / (v - iou2 + 1.0 + _EPS)
    ciou = iou2 - cd / c2 - alpha * v
    loss_box = (npos_raw
                - jnp.sum(wf * ciou, axis=(0, 1), keepdims=True)) / npos

    t_obj = (colsum_wf > 0.0).astype(jnp.float32)
    x = pft[4:5, :]
    bce = jnp.maximum(x, 0.0) - x * t_obj + jnp.log1p(jnp.exp(-jnp.abs(x)))
    loss_obj = jnp.sum(bce, axis=(0, 1), keepdims=True) / float(_N)

    sum_cls = ((1.0 - _CLS_SMOOTH) / _ASSIGN_CLS_W
               * jnp.sum(wf * cc05, axis=(0, 1), keepdims=True)
               - (_CLS_SMOOTH / float(_NC))
               * jnp.sum(colsum_wf * logp_rowsum, axis=(0, 1), keepdims=True))
    loss_cls = sum_cls / npos

    contrib = (_LAMBDA_BOX * loss_box + _LAMBDA_OBJ * loss_obj
               + _LAMBDA_CLS * loss_cls) / 8.0

    @pl.when(b == 0)
    def _():
        out_ref[:, :] = jnp.zeros((1, 1), jnp.float32)

    out_ref[:, :] += contrib


def _make_anchor_const():
    al = []
    sl = []
    for (h, w) in ((80, 80), (40, 40), (20, 20)):
        stride = 640.0 / max(h, w)
        sy, sx = np.meshgrid(np.arange(h), np.arange(w), indexing='ij')
        al.append(np.stack([sx, sy], axis=-1).astype(np.float32).reshape(-1, 2))
        sl.append(np.full((h * w,), stride, dtype=np.float32))
    anchors = np.concatenate(al, axis=0)
    strides = np.concatenate(sl, axis=0)
    anc = np.zeros((8, _N), dtype=np.float32)
    anc[0] = anchors[:, 0]
    anc[1] = anchors[:, 1]
    anc[2] = strides
    anc[3] = (anchors[:, 0] + 0.5) * strides
    anc[4] = (anchors[:, 1] + 0.5) * strides
    anc[5] = np.log(strides)
    anc[6] = _CENTER_COST_W / strides
    return anc


_ANC_NP = _make_anchor_const()


def kernel(p3, p4, p5, gt_boxes, gt_labels):
    B = p3.shape[0]
    f3 = p3.reshape(B, -1, 5 + _NC)
    f4 = p4.reshape(B, -1, 5 + _NC)
    f5 = p5.reshape(B, -1, 5 + _NC)

    gb = gt_boxes * _IMG
    gx1 = gb[..., 0] - 0.5 * gb[..., 2]
    gy1 = gb[..., 1] - 0.5 * gb[..., 3]
    gx2 = gb[..., 0] + 0.5 * gb[..., 2]
    gy2 = gb[..., 1] + 0.5 * gb[..., 3]
    labf = gt_labels.astype(jnp.float32)
    zero = jnp.zeros_like(labf)
    gtf = jnp.stack([gx1, gy1, gx2, gy2, labf, zero, zero, zero], axis=-1)

    out = pl.pallas_call(
        _loss_body,
        grid=(B,),
        in_specs=[
            pl.BlockSpec((8, _N), lambda b: (0, 0)),
            pl.BlockSpec((1, 6400, 85), lambda b: (b, 0, 0)),
            pl.BlockSpec((1, 1600, 85), lambda b: (b, 0, 0)),
            pl.BlockSpec((1, 400, 85), lambda b: (b, 0, 0)),
            pl.BlockSpec((1, _M, 8), lambda b: (b, 0, 0)),
        ],
        out_specs=pl.BlockSpec((1, 1), lambda b: (0, 0)),
        out_shape=jax.ShapeDtypeStruct((1, 1), jnp.float32),
    )(jnp.asarray(_ANC_NP), f3, f4, f5, gtf)
    return out[0, 0]


# speedup vs baseline: 11.4468x; 1.0031x over previous
import math

import jax
import jax.numpy as jnp
import numpy as np
from jax.experimental import pallas as pl

_NC = 80
_IMG = 640.0
_LAMBDA_BOX = 5.0
_LAMBDA_OBJ = 1.0
_LAMBDA_CLS = 0.5
_ASSIGN_CLS_W = 0.5
_CENTER_RADIUS = 2.0
_TOPK = 20
_CLS_SMOOTH = 0.05
_AREA_MIN = 4.0 / 1.25
_AREA_MAX = 256.0 * 1.25
_SIZE_PRIOR_W = 0.2
_AR_PRIOR_W = 0.1
_IOU_COST_W = 3.0
_CENTER_COST_W = 0.5
_EPS = 1e-7
_N = 8400
_M = 40


def _atan_pos(x):
    inv = x > 1.0
    z = jnp.where(inv, 1.0 / x, x)
    red = z > 0.4142135623730951
    w = jnp.where(red, (z - 1.0) / (z + 1.0), z)
    w2 = w * w
    p = w * (1.0 + w2 * (-1.0 / 3.0 + w2 * (1.0 / 5.0 + w2 * (
        -1.0 / 7.0 + w2 * (1.0 / 9.0 + w2 * (-1.0 / 11.0))))))
    p = jnp.where(red, 0.7853981633974483 + p, p)
    return jnp.where(inv, 1.5707963267948966 - p, p)


def _softplus(x):
    return jnp.maximum(x, 0.0) + jnp.log1p(jnp.exp(-jnp.abs(x)))


def _loss_body(anc_ref, p3_ref, p4_ref, p5_ref, gtf_ref, out_ref):
    b = pl.program_id(0)

    agx = anc_ref[0:1, :]
    agy = anc_ref[1:2, :]
    s = anc_ref[2:3, :]
    apx = anc_ref[3:4, :]
    apy = anc_ref[4:5, :]
    ls = anc_ref[5:6, :]
    inv_s = anc_ref[6:7, :]

    nat = jnp.concatenate([p3_ref[0], p4_ref[0], p5_ref[0]], axis=0)
    pft = jnp.transpose(nat, (1, 0))

    braw = pft[0:4, :]
    sx = jax.nn.sigmoid(braw[0:1, :])
    sy = jax.nn.sigmoid(braw[1:2, :])
    pcx = (sx * 2.0 - 0.5 + agx) * s
    pcy = (sy * 2.0 - 0.5 + agy) * s
    pw = _softplus(braw[2:3, :]) * s
    ph = _softplus(braw[3:4, :]) * s
    px1 = pcx - 0.5 * pw
    py1 = pcy - 0.5 * ph
    px2 = pcx + 0.5 * pw
    py2 = pcy + 0.5 * ph

    cls = pft[5:85, :]
    cmax = jnp.max(cls, axis=0, keepdims=True)
    sse = jnp.sum(jnp.exp(cls - cmax), axis=0, keepdims=True)
    logz = jnp.log(sse) + cmax
    logp_rowsum = jnp.sum(cls, axis=0, keepdims=True) - float(_NC) * logz

    gtf = gtf_ref[0]
    gx1 = gtf[:, 0:1]
    gy1 = gtf[:, 1:2]
    gx2 = gtf[:, 2:3]
    gy2 = gtf[:, 3:4]
    labf = gtf[:, 4:5]
    gcx = (gx1 + gx2) * 0.5
    gcy = (gy1 + gy2) * 0.5
    gw = jnp.maximum(gx2 - gx1, _EPS)
    gh = jnp.maximum(gy2 - gy1, _EPS)

    cls_iota = jax.lax.broadcasted_iota(
        jnp.int32, (_M, _NC), 1).astype(jnp.float32)
    onehot = jnp.where(cls_iota == labf, _ASSIGN_CLS_W, 0.0)
    selcls05 = jnp.dot(onehot, cls, preferred_element_type=jnp.float32)
    cc05 = _ASSIGN_CLS_W * logz - selcls05

    ix1 = jnp.maximum(px1, gx1)
    iy1 = jnp.maximum(py1, gy1)
    ix2 = jnp.minimum(px2, gx2)
    iy2 = jnp.minimum(py2, gy2)
    inter = jnp.maximum(ix2 - ix1, 0.0) * jnp.maximum(iy2 - iy1, 0.0)
    pa = jnp.maximum(px2 - px1, 0.0) * jnp.maximum(py2 - py1, 0.0)
    ga = jnp.maximum(gx2 - gx1, 0.0) * jnp.maximum(gy2 - gy1, 0.0)
    iou = inter / (pa + ga - inter + _EPS)

    in_box = (apx > gx1) & (apx < gx2) & (apy > gy1) & (apy < gy2)
    r = _CENTER_RADIUS * s
    in_ctr = (jnp.abs(apx - gcx) < r) & (jnp.abs(apy - gcy) < r)
    cand = in_box & in_ctr

    dx = apx - gcx
    dy = apy - gcy
    cdist05 = jnp.sqrt(dx * dx + dy * dy + _EPS) * inv_s

    lga = jnp.log(gw * gh)
    u = lga - 2.0 * ls
    size_prior = _SIZE_PRIOR_W * jnp.maximum(
        jnp.maximum(u - math.log(_AREA_MAX), math.log(_AREA_MIN) - u), 0.0)
    ar_prior = _AR_PRIOR_W * jnp.abs(jnp.log(gw / gh))

    cost = (_IOU_COST_W * (1.0 - iou) + cc05
            + cdist05 + size_prior + ar_prior)

    costw = jnp.where(cand, cost, 3e30)
    mv = jnp.min(costw, axis=1, keepdims=True)
    for _ in range(_TOPK - 1):
        mv = jnp.min(jnp.where(costw > mv, costw, 3e30),
                     axis=1, keepdims=True)

    wf = ((cost <= mv) & (cost < 1e4) & cand).astype(jnp.float32)
    colsum_wf = jnp.sum(wf, axis=0, keepdims=True)
    npos_raw = jnp.sum(colsum_wf, axis=(0, 1), keepdims=True)
    npos = jnp.maximum(npos_raw, 1.0)

    pwc = jnp.maximum(px2 - px1, _EPS)
    phc = jnp.maximum(py2 - py1, _EPS)
    iou2 = iou
    ppx = (px1 + px2) * 0.5
    ppy = (py1 + py2) * 0.5
    cd = (ppx - gcx) ** 2 + (ppy - gcy) ** 2
    cw = ((px2 - px1) + (gx2 - gx1)) - (ix2 - ix1)
    chh = ((py2 - py1) + (gy2 - gy1)) - (iy2 - iy1)
    c2 = cw * cw + chh * chh + _EPS
    v = (4.0 / math.pi ** 2) * (_atan_pos(gw / gh) - _atan_pos(pwc / phc)) ** 2
    alpha = v / (v - iou2 + 1.0 + _EPS)
    ciou = iou2 - cd / c2 - alpha * v
    loss_box = (npos_raw
                - jnp.sum(wf * ciou, axis=(0, 1), keepdims=True)) / npos

    t_obj = (colsum_wf > 0.0).astype(jnp.float32)
    x = pft[4:5, :]
    bce = jnp.maximum(x, 0.0) - x * t_obj + jnp.log1p(jnp.exp(-jnp.abs(x)))
    loss_obj = jnp.sum(bce, axis=(0, 1), keepdims=True) / float(_N)

    sum_cls = ((1.0 - _CLS_SMOOTH) / _ASSIGN_CLS_W
               * jnp.sum(wf * cc05, axis=(0, 1), keepdims=True)
               - (_CLS_SMOOTH / float(_NC))
               * jnp.sum(colsum_wf * logp_rowsum, axis=(0, 1), keepdims=True))
    loss_cls = sum_cls / npos

    contrib = (_LAMBDA_BOX * loss_box + _LAMBDA_OBJ * loss_obj
               + _LAMBDA_CLS * loss_cls) / 8.0

    @pl.when(b == 0)
    def _():
        out_ref[:, :] = jnp.zeros((1, 1), jnp.float32)

    out_ref[:, :] += contrib


def _make_anchor_const():
    al = []
    sl = []
    for (h, w) in ((80, 80), (40, 40), (20, 20)):
        stride = 640.0 / max(h, w)
        sy, sx = np.meshgrid(np.arange(h), np.arange(w), indexing='ij')
        al.append(np.stack([sx, sy], axis=-1).astype(np.float32).reshape(-1, 2))
        sl.append(np.full((h * w,), stride, dtype=np.float32))
    anchors = np.concatenate(al, axis=0)
    strides = np.concatenate(sl, axis=0)
    anc = np.zeros((8, _N), dtype=np.float32)
    anc[0] = anchors[:, 0]
    anc[1] = anchors[:, 1]
    anc[2] = strides
    anc[3] = (anchors[:, 0] + 0.5) * strides
    anc[4] = (anchors[:, 1] + 0.5) * strides
    anc[5] = np.log(strides)
    anc[6] = _CENTER_COST_W / strides
    return anc


_ANC_NP = _make_anchor_const()


def kernel(p3, p4, p5, gt_boxes, gt_labels):
    B = p3.shape[0]
    f3 = p3.reshape(B, -1, 5 + _NC)
    f4 = p4.reshape(B, -1, 5 + _NC)
    f5 = p5.reshape(B, -1, 5 + _NC)

    gb = gt_boxes * _IMG
    gx1 = gb[..., 0] - 0.5 * gb[..., 2]
    gy1 = gb[..., 1] - 0.5 * gb[..., 3]
    gx2 = gb[..., 0] + 0.5 * gb[..., 2]
    gy2 = gb[..., 1] + 0.5 * gb[..., 3]
    labf = gt_labels.astype(jnp.float32)
    zero = jnp.zeros_like(labf)
    gtf = jnp.stack([gx1, gy1, gx2, gy2, labf, zero, zero, zero], axis=-1)

    out = pl.pallas_call(
        _loss_body,
        grid=(B,),
        in_specs=[
            pl.BlockSpec((8, _N), lambda b: (0, 0)),
            pl.BlockSpec((1, 6400, 85), lambda b: (b, 0, 0)),
            pl.BlockSpec((1, 1600, 85), lambda b: (b, 0, 0)),
            pl.BlockSpec((1, 400, 85), lambda b: (b, 0, 0)),
            pl.BlockSpec((1, _M, 8), lambda b: (b, 0, 0)),
        ],
        out_specs=pl.BlockSpec((1, 1), lambda b: (0, 0)),
        out_shape=jax.ShapeDtypeStruct((1, 1), jnp.float32),
    )(jnp.asarray(_ANC_NP), f3, f4, f5, gtf)
    return out[0, 0]
